# baseline (speedup 1.0000x reference)
import functools

import jax
import jax.numpy as jnp
from jax import lax
from jax.experimental import pallas as pl
from jax.experimental.pallas import tpu as pltpu
from jax.experimental.pallas import tpu_sc as plsc

N_NODES = 10000
N_EDGES = 320000
D = 128

NC = 2
NS = 16
NW = NC * NS

K = 40
EP = N_EDGES // NW
CH = EP // K
KO = 80
RCH = N_NODES // KO

NBUF = 6
LA = 4


def _sc_segment_sum(ei, nfeats, zrows):
  mesh = plsc.VectorSubcoreMesh(core_axis_name="c", subcore_axis_name="s")

  @functools.partial(
      pl.kernel,
      mesh=mesh,
      out_type=jax.ShapeDtypeStruct((NC, N_NODES, D), jnp.float32),
      scratch_types=[
          pltpu.VMEM((NBUF, 2, K), jnp.int32),
          pltpu.VMEM((NBUF, K, D), jnp.float32),
          pltpu.VMEM_SHARED((N_NODES, D), jnp.float32),
          pltpu.SemaphoreType.DMA((NBUF,)),
          pltpu.SemaphoreType.DMA((NBUF,)),
          pltpu.SemaphoreType.DMA((NBUF,)),
          pltpu.SemaphoreType.DMA,
      ],
  )
  def k(ei_hbm, nf_hbm, z_hbm, out_hbm, idx_v, bufs, acc, isem, gsem, ssem,
        zsem):
    c = lax.axis_index("c")
    s = lax.axis_index("s")
    wid = s * NC + c

    nz = (RCH - s + NS - 1) // NS

    def zissue(i, _):
      t = s + i * NS
      pltpu.async_copy(z_hbm, acc.at[pl.ds(t * KO, KO)], zsem)
      return 0

    def zwait(i, _):
      pltpu.make_async_copy(z_hbm, acc.at[pl.ds(s * KO, KO)], zsem).wait()
      return 0

    lax.fori_loop(0, nz, zissue, 0)
    lax.fori_loop(0, nz, zwait, 0)
    plsc.subcore_barrier()

    def issue_idx(j, b):
      pltpu.async_copy(ei_hbm.at[wid].at[j], idx_v.at[b], isem.at[b])

    def wait_idx(j, b):
      pltpu.make_async_copy(ei_hbm.at[wid].at[j], idx_v.at[b],
                            isem.at[b]).wait()

    def issue_gather(j, b):
      pltpu.async_copy(nf_hbm.at[idx_v.at[b].at[0]], bufs.at[b], gsem.at[b])

    def wait_gather(j, b):
      pltpu.make_async_copy(nf_hbm.at[idx_v.at[b].at[0]], bufs.at[b],
                            gsem.at[b]).wait()

    def issue_scatter(j, b):
      pltpu.async_copy(bufs.at[b], acc.at[idx_v.at[b].at[1]], ssem.at[b],
                       add=True)

    def wait_scatter(j, b):
      pltpu.make_async_copy(bufs.at[b], acc.at[idx_v.at[b].at[1]],
                            ssem.at[b]).wait()

    for j in range(LA):
      issue_idx(j, j % NBUF)
    for j in range(LA):
      wait_idx(j, j % NBUF)
      issue_gather(j, j % NBUF)
    for j in range(NBUF - LA):
      bl = (j + LA) % NBUF
      issue_idx(j + LA, bl)
      wait_gather(j, j % NBUF)
      issue_scatter(j, j % NBUF)
      wait_idx(j + LA, bl)
      issue_gather(j + LA, bl)

    def step(j, bl, b):
      wait_scatter(j + LA - NBUF, bl)
      issue_idx(j + LA, bl)
      wait_gather(j, b)
      issue_scatter(j, b)
      wait_idx(j + LA, bl)
      issue_gather(j + LA, bl)

    n_steady = (CH - NBUF) // NBUF
    rem = (CH - NBUF) % NBUF

    def body(g, _):
      for bi in range(NBUF):
        j = (NBUF - LA) + g * NBUF + bi
        step(j, (j + LA) % NBUF, j % NBUF)
      return 0

    lax.fori_loop(0, n_steady, body, 0)
    for bi in range(rem):
      j = (NBUF - LA) + n_steady * NBUF + bi
      step(j, (j + LA) % NBUF, j % NBUF)

    for j in range(CH - LA, CH):
      wait_scatter(j + LA - NBUF, (j + LA) % NBUF)
      wait_gather(j, j % NBUF)
      issue_scatter(j, j % NBUF)
    for j in range(CH - NBUF + LA, CH):
      wait_scatter(j, j % NBUF)

    plsc.subcore_barrier()

    def oissue(i, _):
      t = s + i * NS
      pltpu.async_copy(acc.at[pl.ds(t * KO, KO)],
                       out_hbm.at[c].at[pl.ds(t * KO, KO)], zsem)
      return 0

    def owait(i, _):
      pltpu.make_async_copy(acc.at[pl.ds(s * KO, KO)],
                            out_hbm.at[c].at[pl.ds(s * KO, KO)], zsem).wait()
      return 0

    lax.fori_loop(0, nz, oissue, 0)
    lax.fori_loop(0, nz, owait, 0)

  return k(ei, nfeats, zrows)


def _tc_finish(acc, nfeats, wt, b2, scale):
  BR = 1000
  grid = N_NODES // BR

  def body(scale_ref, acc_ref, x_ref, wt_ref, b_ref, o_ref):
    h = x_ref[...] * scale_ref[0] + acc_ref[0] + acc_ref[1]
    o_ref[...] = (
        jnp.dot(h, wt_ref[...], preferred_element_type=jnp.float32) + b_ref[...]
    )

  return pl.pallas_call(
      body,
      grid=(grid,),
      in_specs=[
          pl.BlockSpec(memory_space=pltpu.SMEM),
          pl.BlockSpec((NC, BR, D), lambda i: (0, i, 0)),
          pl.BlockSpec((BR, D), lambda i: (i, 0)),
          pl.BlockSpec((D, D), lambda i: (0, 0)),
          pl.BlockSpec((1, D), lambda i: (0, 0)),
      ],
      out_specs=pl.BlockSpec((BR, D), lambda i: (i, 0)),
      out_shape=jax.ShapeDtypeStruct((N_NODES, D), jnp.float32),
  )(scale, acc, nfeats, wt, b2)


@jax.jit
def kernel(nfeats, edge_index, W, b, alpha):
  ei = edge_index.astype(jnp.int32)
  idx_g = jnp.stack(
      [ei[0].reshape(NW, CH, K), ei[1].reshape(NW, CH, K)], axis=2)
  zrows = jnp.zeros((KO, D), jnp.float32)
  acc = _sc_segment_sum(idx_g, nfeats, zrows)
  scale = (1.0 + alpha).astype(jnp.float32)
  return _tc_finish(acc, nfeats, W.T, b.reshape(1, D), scale)


# speedup vs baseline: 1.4176x; 1.4176x over previous
import functools

import jax
import jax.numpy as jnp
from jax import lax
from jax.experimental import pallas as pl
from jax.experimental.pallas import tpu as pltpu
from jax.experimental.pallas import tpu_sc as plsc

N_NODES = 10000
N_EDGES = 320000
D = 128

NC = 2
NS = 16
NW = NC * NS

K = 80
EP = N_EDGES // NW
CH = EP // K
KO = 1000
RCH = N_NODES // KO

NBUF = 3
LA = 2


def _sc_segment_sum(ei, nfeats, zrows):
  mesh = plsc.VectorSubcoreMesh(core_axis_name="c", subcore_axis_name="s")

  @functools.partial(
      pl.kernel,
      mesh=mesh,
      out_type=jax.ShapeDtypeStruct((NC, N_NODES, D), jnp.float32),
      scratch_types=[
          pltpu.VMEM((NBUF, 2, K), jnp.int32),
          pltpu.VMEM((NBUF, K, D), jnp.float32),
          pltpu.VMEM_SHARED((N_NODES, D), jnp.float32),
          pltpu.SemaphoreType.DMA((NBUF,)),
          pltpu.SemaphoreType.DMA((NBUF,)),
          pltpu.SemaphoreType.DMA((NBUF,)),
          pltpu.SemaphoreType.DMA,
      ],
  )
  def k(ei_hbm, nf_hbm, z_hbm, out_hbm, idx_v, bufs, acc, isem, gsem, ssem,
        zsem):
    c = lax.axis_index("c")
    s = lax.axis_index("s")
    wid = s * NC + c

    nz = (RCH - s + NS - 1) // NS

    def zissue(i, _):
      t = s + i * NS
      pltpu.async_copy(z_hbm, acc.at[pl.ds(t * KO, KO)], zsem)
      return 0

    def zwait(i, _):
      pltpu.make_async_copy(z_hbm, acc.at[pl.ds(s * KO, KO)], zsem).wait()
      return 0

    lax.fori_loop(0, nz, zissue, 0)
    lax.fori_loop(0, nz, zwait, 0)
    plsc.subcore_barrier()

    def issue_idx(j, b):
      pltpu.async_copy(ei_hbm.at[wid].at[j], idx_v.at[b], isem.at[b])

    def wait_idx(j, b):
      pltpu.make_async_copy(ei_hbm.at[wid].at[j], idx_v.at[b],
                            isem.at[b]).wait()

    def issue_gather(j, b):
      pltpu.async_copy(nf_hbm.at[idx_v.at[b].at[0]], bufs.at[b], gsem.at[b])

    def wait_gather(j, b):
      pltpu.make_async_copy(nf_hbm.at[idx_v.at[b].at[0]], bufs.at[b],
                            gsem.at[b]).wait()

    def issue_scatter(j, b):
      pltpu.async_copy(bufs.at[b], acc.at[idx_v.at[b].at[1]], ssem.at[b],
                       add=True)

    def wait_scatter(j, b):
      pltpu.make_async_copy(bufs.at[b], acc.at[idx_v.at[b].at[1]],
                            ssem.at[b]).wait()

    for j in range(LA):
      issue_idx(j, j % NBUF)
    for j in range(LA):
      wait_idx(j, j % NBUF)
      issue_gather(j, j % NBUF)
    for j in range(NBUF - LA):
      bl = (j + LA) % NBUF
      issue_idx(j + LA, bl)
      wait_gather(j, j % NBUF)
      issue_scatter(j, j % NBUF)
      wait_idx(j + LA, bl)
      issue_gather(j + LA, bl)

    def step(j, bl, b):
      wait_scatter(j + LA - NBUF, bl)
      issue_idx(j + LA, bl)
      wait_gather(j, b)
      issue_scatter(j, b)
      wait_idx(j + LA, bl)
      issue_gather(j + LA, bl)

    n_steady = (CH - NBUF) // NBUF
    rem = (CH - NBUF) % NBUF

    def body(g, _):
      for bi in range(NBUF):
        j = (NBUF - LA) + g * NBUF + bi
        step(j, (j + LA) % NBUF, j % NBUF)
      return 0

    lax.fori_loop(0, n_steady, body, 0)
    for bi in range(rem):
      j = (NBUF - LA) + n_steady * NBUF + bi
      step(j, (j + LA) % NBUF, j % NBUF)

    for j in range(CH - LA, CH):
      wait_scatter(j + LA - NBUF, (j + LA) % NBUF)
      wait_gather(j, j % NBUF)
      issue_scatter(j, j % NBUF)
    for j in range(CH - NBUF + LA, CH):
      wait_scatter(j, j % NBUF)

    plsc.subcore_barrier()

    def oissue(i, _):
      t = s + i * NS
      pltpu.async_copy(acc.at[pl.ds(t * KO, KO)],
                       out_hbm.at[c].at[pl.ds(t * KO, KO)], zsem)
      return 0

    def owait(i, _):
      pltpu.make_async_copy(acc.at[pl.ds(s * KO, KO)],
                            out_hbm.at[c].at[pl.ds(s * KO, KO)], zsem).wait()
      return 0

    lax.fori_loop(0, nz, oissue, 0)
    lax.fori_loop(0, nz, owait, 0)

  return k(ei, nfeats, zrows)


def _tc_finish(acc, nfeats, wt, b2, scale):
  BR = 1000
  grid = N_NODES // BR

  def body(scale_ref, acc_ref, x_ref, wt_ref, b_ref, o_ref):
    h = x_ref[...] * scale_ref[0] + acc_ref[0] + acc_ref[1]
    o_ref[...] = (
        jnp.dot(h, wt_ref[...], preferred_element_type=jnp.float32) + b_ref[...]
    )

  return pl.pallas_call(
      body,
      grid=(grid,),
      in_specs=[
          pl.BlockSpec(memory_space=pltpu.SMEM),
          pl.BlockSpec((NC, BR, D), lambda i: (0, i, 0)),
          pl.BlockSpec((BR, D), lambda i: (i, 0)),
          pl.BlockSpec((D, D), lambda i: (0, 0)),
          pl.BlockSpec((1, D), lambda i: (0, 0)),
      ],
      out_specs=pl.BlockSpec((BR, D), lambda i: (i, 0)),
      out_shape=jax.ShapeDtypeStruct((N_NODES, D), jnp.float32),
  )(scale, acc, nfeats, wt, b2)


@jax.jit
def kernel(nfeats, edge_index, W, b, alpha):
  ei = edge_index.astype(jnp.int32)
  idx_g = jnp.stack(
      [ei[0].reshape(NW, CH, K), ei[1].reshape(NW, CH, K)], axis=2)
  zrows = jnp.zeros((KO, D), jnp.float32)
  acc = _sc_segment_sum(idx_g, nfeats, zrows)
  scale = (1.0 + alpha).astype(jnp.float32)
  return _tc_finish(acc, nfeats, W.T, b.reshape(1, D), scale)
